# baseline (speedup 1.0000x reference)
import functools
import math

import jax
import jax.numpy as jnp
from jax import lax
from jax.experimental import pallas as pl
from jax.experimental.pallas import tpu as pltpu
from jax.experimental.pallas import tpu_sc as plsc

DIM = 64
SCALE = math.sqrt(DIM)
LANES = 16
NC, NS = 2, 16
NW = NC * NS

SEQ = 200
NTOK = 4096
TB = 128
NBUF = 2


def _make_kernel():
    n_blocks_total = SEQ * (NTOK // TB)
    n_blocks = n_blocks_total // NW
    tpj = NTOK // TB
    mesh = plsc.VectorSubcoreMesh(core_axis_name="c", subcore_axis_name="s")

    @functools.partial(
        pl.kernel,
        out_type=jax.ShapeDtypeStruct((SEQ, 8, tpj, 8, TB), jnp.float32),
        mesh=mesh,
        scratch_types=[
            pltpu.VMEM((n_blocks, TB), jnp.int32),
            pltpu.VMEM((NBUF, TB), jnp.int32),
            pltpu.VMEM((NBUF, TB), jnp.int32),
            pltpu.VMEM((NBUF, TB, 2 * DIM), jnp.float32),
            pltpu.VMEM((NBUF, 8, 8, TB), jnp.float32),
            [pltpu.SemaphoreType.DMA] * NBUF,
            [pltpu.SemaphoreType.DMA] * NBUF,
        ],
        compiler_params=pltpu.CompilerParams(
            use_tc_tiling_on_sc=False, needs_layout_passes=False
        ),
    )
    def emb_kernel(x_hbm, w_hbm, out_hbm, idx_all, pi_v, h64_v, prow_v,
                   ob_v, gsem, osem):
        wid = lax.axis_index("s") * NC + lax.axis_index("c")
        blk0 = wid * n_blocks

        pltpu.sync_copy(x_hbm.at[pl.ds(blk0, n_blocks)], idx_all)

        def prep_and_fire(i, b):
            for c in range(TB // LANES):
                sl = pl.ds(c * LANES, LANES)
                t = idx_all[i, sl]
                pi_v[b, sl] = lax.shift_right_logical(t, 1)
                h64_v[b, sl] = lax.shift_left(jnp.bitwise_and(t, 1), 6)
            pltpu.async_copy(w_hbm.at[pi_v.at[b]], prow_v.at[b], gsem[b])

        def wait_gather(b):
            pltpu.make_async_copy(
                w_hbm.at[pi_v.at[b]], prow_v.at[b], gsem[b]
            ).wait()

        def transpose_scale(b):
            def body(q):
                c = jnp.bitwise_and(q, 7)
                k = lax.shift_right_logical(q, 3)
                csl = pl.ds(c * LANES, LANES)
                ii = lax.iota(jnp.int32, LANES) + c * LANES
                off = h64_v[b, csl] + k
                val = plsc.load_gather(prow_v.at[b], [ii, off])
                g = lax.shift_right_logical(q, 6)
                kk = jnp.bitwise_and(lax.shift_right_logical(q, 3), 7)
                ob_v[b, g, kk, csl] = val * SCALE

            plsc.parallel_loop(0, 8 * DIM, 1, unroll=8)(body)

        def fire_stores(i, b):
            bid = blk0 + i
            j = lax.shift_right_logical(bid, 5)
            t = jnp.bitwise_and(bid, tpj - 1)
            for g in range(8):
                pltpu.async_copy(
                    ob_v.at[b].at[g], out_hbm.at[j, g, t], osem[b]
                )

        def wait_stores(i, b):
            bid = blk0 + i
            j = lax.shift_right_logical(bid, 5)
            t = jnp.bitwise_and(bid, tpj - 1)
            for g in range(8):
                pltpu.make_async_copy(
                    ob_v.at[b].at[g], out_hbm.at[j, g, t], osem[b]
                ).wait()

        prep_and_fire(0, 0)

        def super_body(s, _):
            for b in range(NBUF):
                i = s * NBUF + b
                ni = i + 1
                nb = (b + 1) % NBUF

                @pl.when(jnp.logical_and(ni >= NBUF, ni < n_blocks))
                def _():
                    wait_stores(ni - NBUF, nb)

                @pl.when(ni < n_blocks)
                def _():
                    prep_and_fire(ni, nb)

                wait_gather(b)
                transpose_scale(b)
                fire_stores(i, b)
            return ()

        lax.fori_loop(0, n_blocks // NBUF, super_body, ())

        for b in range(NBUF):
            wait_stores(n_blocks - NBUF + b, b)

    return emb_kernel


@jax.jit
def kernel(x, W):
    xT = x.T.reshape(SEQ * NTOK // TB, TB)
    W2 = W.reshape(W.shape[0] // 2, 2 * DIM)
    out5 = _make_kernel()(xT, W2)
    return out5.transpose(2, 4, 0, 1, 3).reshape(NTOK, SEQ, DIM)


# speedup vs baseline: 1.0963x; 1.0963x over previous
import functools
import math

import jax
import jax.numpy as jnp
from jax import lax
from jax.experimental import pallas as pl
from jax.experimental.pallas import tpu as pltpu
from jax.experimental.pallas import tpu_sc as plsc

DIM = 64
SCALE = math.sqrt(DIM)
LANES = 16
NC, NS = 2, 16
NW = NC * NS

SEQ = 200
NTOK = 4096
TB = 128
NBUF = 2


def _make_kernel():
    n_blocks_total = SEQ * (NTOK // TB)
    n_blocks = n_blocks_total // NW
    tpj = NTOK // TB
    mesh = plsc.VectorSubcoreMesh(core_axis_name="c", subcore_axis_name="s")

    @functools.partial(
        pl.kernel,
        out_type=jax.ShapeDtypeStruct((SEQ, 8, tpj, 8, TB), jnp.float32),
        mesh=mesh,
        scratch_types=[
            pltpu.VMEM((n_blocks, TB), jnp.int32),
            pltpu.VMEM((NBUF, TB), jnp.int32),
            pltpu.VMEM((NBUF, TB), jnp.int32),
            pltpu.VMEM((NBUF, TB, 2 * DIM), jnp.float32),
            pltpu.VMEM((NBUF, 8, 8, TB), jnp.float32),
            [pltpu.SemaphoreType.DMA] * NBUF,
            [pltpu.SemaphoreType.DMA] * NBUF,
        ],
        compiler_params=pltpu.CompilerParams(
            use_tc_tiling_on_sc=False, needs_layout_passes=False
        ),
    )
    def emb_kernel(x_hbm, w_hbm, out_hbm, idx_all, pi_v, h64_v, prow_v,
                   ob_v, gsem, osem):
        wid = lax.axis_index("s") * NC + lax.axis_index("c")
        blk0 = wid * n_blocks

        pltpu.sync_copy(x_hbm.at[pl.ds(blk0, n_blocks)], idx_all)

        def prep_and_fire(i, b):
            for c in range(TB // LANES):
                sl = pl.ds(c * LANES, LANES)
                t = idx_all[i, sl]
                pi_v[b, sl] = lax.shift_right_logical(t, 1)
                h64_v[b, sl] = lax.shift_left(jnp.bitwise_and(t, 1), 6)
            pltpu.async_copy(w_hbm.at[pi_v.at[b]], prow_v.at[b], gsem[b])

        def wait_gather(b):
            pltpu.make_async_copy(
                w_hbm.at[pi_v.at[b]], prow_v.at[b], gsem[b]
            ).wait()

        def transpose_scale(b):
            def cbody(c):
                csl = pl.ds(c * LANES, LANES)
                h64c = h64_v[b, csl]
                ii = lax.iota(jnp.int32, LANES) + c * LANES
                for k in range(DIM):
                    val = plsc.load_gather(prow_v.at[b], [ii, h64c + k])
                    ob_v[b, k // 8, k % 8, csl] = val * SCALE

            plsc.parallel_loop(0, TB // LANES, 1)(cbody)

        def fire_stores(i, b):
            bid = blk0 + i
            j = lax.shift_right_logical(bid, 5)
            t = jnp.bitwise_and(bid, tpj - 1)
            for g in range(8):
                pltpu.async_copy(
                    ob_v.at[b].at[g], out_hbm.at[j, g, t], osem[b]
                )

        def wait_stores(i, b):
            bid = blk0 + i
            j = lax.shift_right_logical(bid, 5)
            t = jnp.bitwise_and(bid, tpj - 1)
            for g in range(8):
                pltpu.make_async_copy(
                    ob_v.at[b].at[g], out_hbm.at[j, g, t], osem[b]
                ).wait()

        prep_and_fire(0, 0)

        def super_body(s, _):
            for b in range(NBUF):
                i = s * NBUF + b
                ni = i + 1
                nb = (b + 1) % NBUF

                @pl.when(jnp.logical_and(ni >= NBUF, ni < n_blocks))
                def _():
                    wait_stores(ni - NBUF, nb)

                @pl.when(ni < n_blocks)
                def _():
                    prep_and_fire(ni, nb)

                wait_gather(b)
                transpose_scale(b)
                fire_stores(i, b)
            return ()

        lax.fori_loop(0, n_blocks // NBUF, super_body, ())

        for b in range(NBUF):
            wait_stores(n_blocks - NBUF + b, b)

    return emb_kernel


@jax.jit
def kernel(x, W):
    xT = x.T.reshape(SEQ * NTOK // TB, TB)
    W2 = W.reshape(W.shape[0] // 2, 2 * DIM)
    out5 = _make_kernel()(xT, W2)
    return out5.transpose(2, 4, 0, 1, 3).reshape(NTOK, SEQ, DIM)


# speedup vs baseline: 1.1311x; 1.0317x over previous
import functools
import math

import jax
import jax.numpy as jnp
from jax import lax
from jax.experimental import pallas as pl
from jax.experimental.pallas import tpu as pltpu
from jax.experimental.pallas import tpu_sc as plsc

DIM = 64
SCALE = math.sqrt(DIM)
LANES = 16
NC, NS = 2, 16
NW = NC * NS

TB = 128
NBUF = 2


def _make_kernel(n_tokens):
    assert n_tokens % (NW * TB * NBUF) == 0
    tok_pw = n_tokens // NW
    n_chunks = tok_pw // TB
    mesh = plsc.VectorSubcoreMesh(core_axis_name="c", subcore_axis_name="s")

    @functools.partial(
        pl.kernel,
        out_type=jax.ShapeDtypeStruct((n_tokens // 2, 2 * DIM), jnp.float32),
        mesh=mesh,
        scratch_types=[
            pltpu.VMEM((n_chunks, TB), jnp.int32),
            pltpu.VMEM((NBUF, TB), jnp.int32),
            pltpu.VMEM((NBUF, TB), jnp.int32),
            pltpu.VMEM((NBUF, TB, 2 * DIM), jnp.float32),
            pltpu.VMEM((NBUF, TB // 2, 2 * DIM), jnp.float32),
            [pltpu.SemaphoreType.DMA] * NBUF,
            [pltpu.SemaphoreType.DMA] * NBUF,
        ],
        compiler_params=pltpu.CompilerParams(
            use_tc_tiling_on_sc=False, needs_layout_passes=False
        ),
    )
    def emb_kernel(x_hbm, w_hbm, out_hbm, idx_all, pi_v, h64_v, prow_v,
                   emb_v, gsem, osem):
        wid = lax.axis_index("s") * NC + lax.axis_index("c")
        chunk0 = wid * n_chunks

        pltpu.sync_copy(x_hbm.at[pl.ds(chunk0, n_chunks)], idx_all)

        def prep_and_fire(i, b):
            for c in range(TB // LANES):
                sl = pl.ds(c * LANES, LANES)
                t = idx_all[i, sl]
                pi_v[b, sl] = lax.shift_right_logical(t, 1)
                h64_v[b, sl] = lax.shift_left(jnp.bitwise_and(t, 1), 6)
            pltpu.async_copy(w_hbm.at[pi_v.at[b]], prow_v.at[b], gsem[b])

        def wait_gather(b):
            pltpu.make_async_copy(
                w_hbm.at[pi_v.at[b]], prow_v.at[b], gsem[b]
            ).wait()

        def select_scale(b):
            def gbody(g):
                hvec = h64_v[b, pl.ds(g * LANES, LANES)]
                for l in range(LANES):
                    t = g * LANES + l
                    h = hvec[l]
                    r = g * (LANES // 2) + l // 2
                    o64 = (l % 2) * DIM
                    for j in range(DIM // LANES):
                        v = prow_v[b, t, pl.ds(h + j * LANES, LANES)]
                        emb_v[b, r, pl.ds(o64 + j * LANES, LANES)] = (
                            v * SCALE
                        )

            plsc.parallel_loop(0, TB // LANES, 1)(gbody)

        def fire_store(i, b):
            row = (chunk0 + i) * (TB // 2)
            pltpu.async_copy(
                emb_v.at[b], out_hbm.at[pl.ds(row, TB // 2)], osem[b]
            )

        def wait_store(i, b):
            row = (chunk0 + i) * (TB // 2)
            pltpu.make_async_copy(
                emb_v.at[b], out_hbm.at[pl.ds(row, TB // 2)], osem[b]
            ).wait()

        prep_and_fire(0, 0)

        def super_body(s, _):
            for b in range(NBUF):
                i = s * NBUF + b
                ni = i + 1
                nb = (b + 1) % NBUF

                @pl.when(jnp.logical_and(ni >= NBUF, ni < n_chunks))
                def _():
                    wait_store(ni - NBUF, nb)

                @pl.when(ni < n_chunks)
                def _():
                    prep_and_fire(ni, nb)

                wait_gather(b)
                select_scale(b)
                fire_store(i, b)
            return ()

        lax.fori_loop(0, n_chunks // NBUF, super_body, ())

        for b in range(NBUF):
            wait_store(n_chunks - NBUF + b, b)

    return emb_kernel


@jax.jit
def kernel(x, W):
    n_tokens = x.shape[0] * x.shape[1]
    xf = x.reshape(n_tokens // TB, TB)
    W2 = W.reshape(W.shape[0] // 2, 2 * DIM)
    out2 = _make_kernel(n_tokens)(xf, W2)
    return out2.reshape(x.shape[0], x.shape[1], DIM)


# speedup vs baseline: 1.3031x; 1.1520x over previous
import functools
import math

import jax
import jax.numpy as jnp
from jax import lax
from jax.experimental import pallas as pl
from jax.experimental.pallas import tpu as pltpu
from jax.experimental.pallas import tpu_sc as plsc

DIM = 64
SCALE = math.sqrt(DIM)
LANES = 16
NC, NS = 2, 16
NW = NC * NS

XR = 2
SEQ = 200
NBUF = 2
G_SPLITS = ((0, 128), (128, 72))


def _make_kernel(NROWS):
    assert NROWS % (NW * XR * NBUF) == 0
    n_chunks = NROWS // (NW * XR)
    mesh = plsc.VectorSubcoreMesh(core_axis_name="c", subcore_axis_name="s")

    @functools.partial(
        pl.kernel,
        out_type=jax.ShapeDtypeStruct((NROWS, SEQ, DIM), jnp.float32),
        mesh=mesh,
        scratch_types=[
            pltpu.VMEM((NBUF, XR, SEQ), jnp.int32),
            pltpu.VMEM((NBUF, XR, SEQ, DIM), jnp.float32),
            [pltpu.SemaphoreType.DMA] * NBUF,
            [pltpu.SemaphoreType.DMA] * NBUF,
        ],
        compiler_params=pltpu.CompilerParams(
            use_tc_tiling_on_sc=False, skip_device_barrier=True
        ),
    )
    def emb_kernel(x_hbm, w_hbm, out_hbm, idx_v, rows_v, gsem, ssem):
        wid = lax.axis_index("s") * NC + lax.axis_index("c")
        chunk0 = wid * n_chunks

        def fire_gathers(ci, b):
            row = (chunk0 + ci) * XR
            pltpu.sync_copy(x_hbm.at[pl.ds(row, XR)], idx_v.at[b])
            for i in range(XR):
                for off, n in G_SPLITS:
                    pltpu.async_copy(
                        w_hbm.at[idx_v.at[b].at[i, pl.ds(off, n)]],
                        rows_v.at[b].at[i].at[pl.ds(off, n)],
                        gsem[b],
                    )

        def wait_gathers(b):
            for i in range(XR):
                for off, n in G_SPLITS:
                    pltpu.make_async_copy(
                        w_hbm.at[idx_v.at[b].at[i, pl.ds(off, n)]],
                        rows_v.at[b].at[i].at[pl.ds(off, n)],
                        gsem[b],
                    ).wait()

        def fire_store(ci, b):
            row = (chunk0 + ci) * XR
            pltpu.async_copy(rows_v.at[b], out_hbm.at[pl.ds(row, XR)], ssem[b])

        def wait_store(ci, b):
            row = (chunk0 + ci) * XR
            pltpu.make_async_copy(
                rows_v.at[b], out_hbm.at[pl.ds(row, XR)], ssem[b]
            ).wait()

        def scale(b):
            def scale_body(r):
                for i in range(XR):
                    for j in range(DIM // LANES):
                        sl = pl.ds(j * LANES, LANES)
                        rows_v[b, i, r, sl] = rows_v[b, i, r, sl] * SCALE

            plsc.parallel_loop(0, SEQ, 1, unroll=8)(scale_body)

        fire_gathers(0, 0)

        def super_body(s, _):
            for b in range(NBUF):
                ci = s * NBUF + b
                nci = ci + 1
                nb = (b + 1) % NBUF

                @pl.when(jnp.logical_and(nci >= NBUF, nci < n_chunks))
                def _():
                    wait_store(nci - NBUF, nb)

                @pl.when(nci < n_chunks)
                def _():
                    fire_gathers(nci, nb)

                wait_gathers(b)
                scale(b)
                fire_store(ci, b)
            return ()

        lax.fori_loop(0, n_chunks // NBUF, super_body, ())

        for b in range(NBUF):
            wait_store(n_chunks - NBUF + b, b)

    return emb_kernel


@jax.jit
def kernel(x, W):
    return _make_kernel(x.shape[0])(x, W)
